# Initial kernel scaffold; baseline (speedup 1.0000x reference)
import jax
import jax.numpy as jnp
from jax.experimental import pallas as pl


def kernel(x, char_table, pos_table):
    raise NotImplementedError("write your pallas kernel here")


# speedup vs baseline: 4.1560x; 4.1560x over previous
import jax
import jax.numpy as jnp
from jax import lax
from jax.experimental import pallas as pl
from jax.experimental.pallas import tpu as pltpu
from jax.experimental.pallas import tpu_sc as plsc

B, T, V, D = 16384, 3, 10, 10
NC, NS = 2, 16
NW = NC * NS
BPW = B // NW
GROUPS = BPW // 16


def _body(x_hbm, char_hbm, pos_hbm, out_hbm, x_v, char_v, pos_v, ct_v, out_v):
    wid = lax.axis_index("s") * NC + lax.axis_index("c")
    b0 = wid * BPW

    pltpu.sync_copy(x_hbm.at[pl.ds(b0, BPW)], x_v)
    pltpu.sync_copy(char_hbm, char_v)
    pltpu.sync_copy(pos_hbm, pos_v)

    lane = lax.iota(jnp.int32, 16)

    for c0, width in ((0, 16), (16, T * V - 16)):
        c_vec = lane + c0
        mask_c = lane < width
        t_vec = ((c_vec >= V).astype(jnp.int32)
                 + (c_vec >= 2 * V).astype(jnp.int32))
        v_vec = c_vec - t_vec * V
        for d in range(D):
            d_vec = jnp.full((16,), d, jnp.int32)
            cv = plsc.load_gather(char_v, [v_vec, d_vec], mask=mask_c)
            pv = plsc.load_gather(pos_v, [t_vec, d_vec], mask=mask_c)
            plsc.store_scatter(ct_v, [d_vec, c_vec], cv + pv, mask=mask_c)

    def step(i, carry):
        lb = i * 16 + lane
        for t in range(T):
            t_vec = jnp.full((16,), t, jnp.int32)
            xv = plsc.load_gather(x_v, [lb, t_vec])
            cidx = xv + t * V
            for d in range(D):
                d_vec = jnp.full((16,), d, jnp.int32)
                col_vec = jnp.full((16,), t * D + d, jnp.int32)
                val = plsc.load_gather(ct_v, [d_vec, cidx])
                plsc.store_scatter(out_v, [lb, col_vec], val)
        return carry

    lax.fori_loop(0, GROUPS, step, 0)

    pltpu.sync_copy(out_v, out_hbm.at[pl.ds(b0, BPW)])


def kernel(x, char_table, pos_table):
    mesh = plsc.VectorSubcoreMesh(
        core_axis_name="c", subcore_axis_name="s",
        num_cores=NC, num_subcores=NS)
    k = pl.kernel(
        _body,
        out_type=jax.ShapeDtypeStruct((B, T * D), jnp.float32),
        mesh=mesh,
        scratch_types=[
            pltpu.VMEM((BPW, T), jnp.int32),
            pltpu.VMEM((V, D), jnp.float32),
            pltpu.VMEM((T, D), jnp.float32),
            pltpu.VMEM((D, T * V), jnp.float32),
            pltpu.VMEM((BPW, T * D), jnp.float32),
        ],
        compiler_params=pltpu.CompilerParams(
            needs_layout_passes=False, use_tc_tiling_on_sc=False),
    )
    return k(x, char_table, pos_table).reshape(B, T, D)


# speedup vs baseline: 4.2025x; 1.0112x over previous
import jax
import jax.numpy as jnp
from jax import lax
from jax.experimental import pallas as pl
from jax.experimental.pallas import tpu as pltpu
from jax.experimental.pallas import tpu_sc as plsc

B, T, V, D = 16384, 3, 10, 10
NC, NS = 2, 16
NW = NC * NS
BPW = B // NW
GROUPS = BPW // 16


def _body(x_hbm, char_hbm, pos_hbm, out_hbm, x_v, char_v, pos_v, ct_v, out_v):
    wid = lax.axis_index("s") * NC + lax.axis_index("c")
    b0 = wid * BPW

    pltpu.sync_copy(x_hbm.at[pl.ds(b0, BPW)], x_v)
    pltpu.sync_copy(char_hbm, char_v)
    pltpu.sync_copy(pos_hbm, pos_v)

    lane = lax.iota(jnp.int32, 16)

    for c0, width in ((0, 16), (16, T * V - 16)):
        c_vec = lane + c0
        mask_c = lane < width
        t_vec = ((c_vec >= V).astype(jnp.int32)
                 + (c_vec >= 2 * V).astype(jnp.int32))
        v_vec = c_vec - t_vec * V
        for d in range(D):
            d_vec = jnp.full((16,), d, jnp.int32)
            cv = plsc.load_gather(char_v, [v_vec, d_vec], mask=mask_c)
            pv = plsc.load_gather(pos_v, [t_vec, d_vec], mask=mask_c)
            plsc.store_scatter(ct_v, [d_vec, c_vec], cv + pv, mask=mask_c)

    @plsc.parallel_loop(0, GROUPS, step=1, unroll=2)
    def _loop(i):
        lb = i * 16 + lane
        for t in range(T):
            t_vec = jnp.full((16,), t, jnp.int32)
            xv = plsc.load_gather(x_v, [lb, t_vec])
            cidx = xv + t * V
            for d in range(D):
                d_vec = jnp.full((16,), d, jnp.int32)
                col_vec = jnp.full((16,), t * D + d, jnp.int32)
                val = plsc.load_gather(ct_v, [d_vec, cidx])
                plsc.store_scatter(out_v, [lb, col_vec], val)

    pltpu.sync_copy(out_v, out_hbm.at[pl.ds(b0, BPW)])


def kernel(x, char_table, pos_table):
    mesh = plsc.VectorSubcoreMesh(
        core_axis_name="c", subcore_axis_name="s",
        num_cores=NC, num_subcores=NS)
    k = pl.kernel(
        _body,
        out_type=jax.ShapeDtypeStruct((B, T * D), jnp.float32),
        mesh=mesh,
        scratch_types=[
            pltpu.VMEM((BPW, T), jnp.int32),
            pltpu.VMEM((V, D), jnp.float32),
            pltpu.VMEM((T, D), jnp.float32),
            pltpu.VMEM((D, T * V), jnp.float32),
            pltpu.VMEM((BPW, T * D), jnp.float32),
        ],
        compiler_params=pltpu.CompilerParams(
            needs_layout_passes=False, use_tc_tiling_on_sc=False),
    )
    return k(x, char_table, pos_table).reshape(B, T, D)
